# baseline (speedup 1.0000x reference)
import jax
import jax.numpy as jnp
from jax import lax
from jax.experimental import pallas as pl
from jax.experimental.pallas import tpu as pltpu

SIZE = 1000000
DIM = 32
BATCH = 16384

REG = BATCH + 256
TILE_END = 999936
ANCHOR_CAP = TILE_END - REG
EDGE = SIZE - TILE_END


FLAT = DIM * SIZE
COPY_BLOCKS = 25
COPY_CHUNK = FLAT // COPY_BLOCKS


def _copy_body(in_ref, out_ref):
  out_ref[...] = in_ref[...]


def _merge_region(out_ref, regbuf, sem, start, width, rolled, mask):
  rd = pltpu.make_async_copy(
      out_ref.at[:, pl.ds(start, width)], regbuf.at[:, pl.ds(0, width)], sem)
  rd.start()
  rd.wait()
  regbuf[:, pl.ds(0, width)] = jnp.where(
      mask, rolled, regbuf[:, pl.ds(0, width)])
  wr = pltpu.make_async_copy(
      regbuf.at[:, pl.ds(0, width)], out_ref.at[:, pl.ds(start, width)], sem)
  wr.start()
  wr.wait()


def _tc_body(ptr_ref, zT_ref, bank_in_ref, outT_ref, znbuf, regbuf, w_sem):
  del bank_in_ref
  zt = zT_ref[...]
  norm = jnp.sqrt(jnp.sum(zt * zt, axis=0, keepdims=True))
  znbuf[:, pl.ds(0, BATCH)] = zt / jnp.maximum(norm, 1e-12)

  s = jnp.remainder(ptr_ref[0], SIZE)
  lane = lax.broadcasted_iota(jnp.int32, (DIM, REG), 1)
  no_wrap = s + BATCH <= SIZE

  @pl.when(no_wrap)
  def _():
    a = jnp.minimum((s // 128) * 128, ANCHOR_CAP)
    a = pl.multiple_of(a, 128)
    r = s - a
    rolled = pltpu.roll(znbuf[...], r, axis=1)
    mask = jnp.logical_and(lane >= r, lane < r + BATCH)
    _merge_region(outT_ref, regbuf, w_sem, a, REG, rolled, mask)

  @pl.when(jnp.logical_not(no_wrap))
  def _():
    r_t = s - ANCHOR_CAP
    rolled_t = pltpu.roll(znbuf[...], jnp.remainder(r_t, REG), axis=1)
    mask_t = lane >= r_t
    _merge_region(outT_ref, regbuf, w_sem, ANCHOR_CAP, REG, rolled_t, mask_t)
    b1 = s + BATCH - SIZE
    rolled_h = pltpu.roll(znbuf[:, pl.ds(0, BATCH)], b1, axis=1)
    mask_h = lane[:, :BATCH] < b1
    _merge_region(outT_ref, regbuf, w_sem, 0, BATCH, rolled_h, mask_h)

def _edge_body(ptr_ref, zT_ref, in_ref, out_ref):
  s = jnp.remainder(ptr_ref[0], SIZE)
  se = s - TILE_END
  zt = zT_ref[...]
  norm = jnp.sqrt(jnp.sum(zt * zt, axis=0, keepdims=True))
  zn = zt / jnp.maximum(norm, 1e-12)
  rolled = pltpu.roll(zn, jnp.remainder(se, BATCH), axis=1)[:, :128]
  lane = lax.broadcasted_iota(jnp.int32, (DIM, 128), 1)
  mask = jnp.logical_and(lane >= se, lane < se + BATCH)
  out_ref[...] = jnp.where(mask, rolled, in_ref[...])


def kernel(z, bank, ptr):
  zT = z.T
  bankT = bank.T
  bank_flat = bankT.reshape(FLAT)
  copy_flat = pl.pallas_call(
      _copy_body,
      grid=(COPY_BLOCKS,),
      in_specs=[pl.BlockSpec((COPY_CHUNK,), lambda i: (i,))],
      out_specs=pl.BlockSpec((COPY_CHUNK,), lambda i: (i,)),
      out_shape=jax.ShapeDtypeStruct((FLAT,), jnp.float32),
      name="bank_copy",
  )(bank_flat)
  bank_copy = copy_flat.reshape(DIM, SIZE)
  outT = pl.pallas_call(
      _tc_body,
      in_specs=[
          pl.BlockSpec(memory_space=pltpu.SMEM),
          pl.BlockSpec(memory_space=pltpu.VMEM),
          pl.BlockSpec(memory_space=pl.ANY),
      ],
      out_specs=pl.BlockSpec(memory_space=pl.ANY),
      out_shape=jax.ShapeDtypeStruct((DIM, SIZE), jnp.float32),
      input_output_aliases={2: 0},
      scratch_shapes=[
          pltpu.VMEM((DIM, REG), jnp.float32),
          pltpu.VMEM((DIM, REG), jnp.float32),
          pltpu.SemaphoreType.DMA,
      ],
      name="bank_window_write",
  )(ptr, zT, bank_copy)
  outT = pl.pallas_call(
      _edge_body,
      grid=(1,),
      in_specs=[
          pl.BlockSpec(memory_space=pltpu.SMEM),
          pl.BlockSpec((DIM, BATCH), lambda i: (0, 0)),
          pl.BlockSpec((DIM, 128), lambda i: (0, TILE_END // 128)),
      ],
      out_specs=pl.BlockSpec((DIM, 128), lambda i: (0, TILE_END // 128)),
      out_shape=jax.ShapeDtypeStruct((DIM, SIZE), jnp.float32),
      input_output_aliases={2: 0},
      name="bank_edge_fix",
  )(ptr, zT, outT)
  new_bank = outT.T
  p = ptr[0]
  new_ptr = (p + BATCH) % SIZE
  wrapped = jnp.logical_or(new_ptr < p, p + BATCH >= SIZE)
  return new_bank, jnp.array([new_ptr], dtype=jnp.int32), jnp.reshape(wrapped, (1,))


# speedup vs baseline: 65.1603x; 65.1603x over previous
import jax
import jax.numpy as jnp
from jax import lax
from jax.experimental import pallas as pl
from jax.experimental.pallas import tpu as pltpu

SIZE = 1000000
DIM = 32
BATCH = 16384

REG = BATCH + 256
TILE_END = 999936
ANCHOR_CAP = TILE_END - REG
EDGE = SIZE - TILE_END


COPY_W = 64000
COPY_BLOCKS = (SIZE + COPY_W - 1) // COPY_W


def _copy_body(in_ref, out_ref):
  out_ref[...] = in_ref[...]


def _merge_region(out_ref, regbuf, sem, start, width, rolled, mask):
  rd = pltpu.make_async_copy(
      out_ref.at[:, pl.ds(start, width)], regbuf.at[:, pl.ds(0, width)], sem)
  rd.start()
  rd.wait()
  regbuf[:, pl.ds(0, width)] = jnp.where(
      mask, rolled, regbuf[:, pl.ds(0, width)])
  wr = pltpu.make_async_copy(
      regbuf.at[:, pl.ds(0, width)], out_ref.at[:, pl.ds(start, width)], sem)
  wr.start()
  wr.wait()


def _tc_body(ptr_ref, zT_ref, bank_in_ref, outT_ref, znbuf, regbuf, w_sem):
  del bank_in_ref
  zt = zT_ref[...]
  norm = jnp.sqrt(jnp.sum(zt * zt, axis=0, keepdims=True))
  znbuf[:, pl.ds(0, BATCH)] = zt / jnp.maximum(norm, 1e-12)

  s = jnp.remainder(ptr_ref[0], SIZE)
  lane = lax.broadcasted_iota(jnp.int32, (DIM, REG), 1)
  no_wrap = s + BATCH <= SIZE

  @pl.when(no_wrap)
  def _():
    a = jnp.minimum((s // 128) * 128, ANCHOR_CAP)
    a = pl.multiple_of(a, 128)
    r = s - a
    rolled = pltpu.roll(znbuf[...], r, axis=1)
    mask = jnp.logical_and(lane >= r, lane < r + BATCH)
    _merge_region(outT_ref, regbuf, w_sem, a, REG, rolled, mask)

  @pl.when(jnp.logical_not(no_wrap))
  def _():
    r_t = s - ANCHOR_CAP
    rolled_t = pltpu.roll(znbuf[...], jnp.remainder(r_t, REG), axis=1)
    mask_t = lane >= r_t
    _merge_region(outT_ref, regbuf, w_sem, ANCHOR_CAP, REG, rolled_t, mask_t)
    b1 = s + BATCH - SIZE
    rolled_h = pltpu.roll(znbuf[:, pl.ds(0, BATCH)], b1, axis=1)
    mask_h = lane[:, :BATCH] < b1
    _merge_region(outT_ref, regbuf, w_sem, 0, BATCH, rolled_h, mask_h)

def _edge_body(ptr_ref, zT_ref, in_ref, out_ref):
  s = jnp.remainder(ptr_ref[0], SIZE)
  se = s - TILE_END
  zt = zT_ref[...]
  norm = jnp.sqrt(jnp.sum(zt * zt, axis=0, keepdims=True))
  zn = zt / jnp.maximum(norm, 1e-12)
  rolled = pltpu.roll(zn, jnp.remainder(se, BATCH), axis=1)[:, :128]
  lane = lax.broadcasted_iota(jnp.int32, (DIM, 128), 1)
  mask = jnp.logical_and(lane >= se, lane < se + BATCH)
  out_ref[...] = jnp.where(mask, rolled, in_ref[...])


def kernel(z, bank, ptr):
  zT = z.T
  bankT = bank.T
  bank_copy = pl.pallas_call(
      _copy_body,
      grid=(COPY_BLOCKS,),
      in_specs=[pl.BlockSpec((DIM, COPY_W), lambda i: (0, i))],
      out_specs=pl.BlockSpec((DIM, COPY_W), lambda i: (0, i)),
      out_shape=jax.ShapeDtypeStruct((DIM, SIZE), jnp.float32),
      name="bank_copy",
  )(bankT)
  outT = pl.pallas_call(
      _tc_body,
      in_specs=[
          pl.BlockSpec(memory_space=pltpu.SMEM),
          pl.BlockSpec(memory_space=pltpu.VMEM),
          pl.BlockSpec(memory_space=pl.ANY),
      ],
      out_specs=pl.BlockSpec(memory_space=pl.ANY),
      out_shape=jax.ShapeDtypeStruct((DIM, SIZE), jnp.float32),
      input_output_aliases={2: 0},
      scratch_shapes=[
          pltpu.VMEM((DIM, REG), jnp.float32),
          pltpu.VMEM((DIM, REG), jnp.float32),
          pltpu.SemaphoreType.DMA,
      ],
      name="bank_window_write",
  )(ptr, zT, bank_copy)
  outT = pl.pallas_call(
      _edge_body,
      grid=(1,),
      in_specs=[
          pl.BlockSpec(memory_space=pltpu.SMEM),
          pl.BlockSpec((DIM, BATCH), lambda i: (0, 0)),
          pl.BlockSpec((DIM, 128), lambda i: (0, TILE_END // 128)),
      ],
      out_specs=pl.BlockSpec((DIM, 128), lambda i: (0, TILE_END // 128)),
      out_shape=jax.ShapeDtypeStruct((DIM, SIZE), jnp.float32),
      input_output_aliases={2: 0},
      name="bank_edge_fix",
  )(ptr, zT, outT)
  new_bank = outT.T
  p = ptr[0]
  new_ptr = (p + BATCH) % SIZE
  wrapped = jnp.logical_or(new_ptr < p, p + BATCH >= SIZE)
  return new_bank, jnp.array([new_ptr], dtype=jnp.int32), jnp.reshape(wrapped, (1,))


# speedup vs baseline: 65.4755x; 1.0048x over previous
import jax
import jax.numpy as jnp
from jax import lax
from jax.experimental import pallas as pl
from jax.experimental.pallas import tpu as pltpu

SIZE = 1000000
DIM = 32
BATCH = 16384

REG = BATCH + 256
TILE_END = 999936
ANCHOR_CAP = TILE_END - REG
EDGE = SIZE - TILE_END


COPY_W = 102400
COPY_BLOCKS = (SIZE + COPY_W - 1) // COPY_W


def _copy_body(in_ref, out_ref):
  out_ref[...] = in_ref[...]


def _merge_region(out_ref, regbuf, sem, start, width, rolled, mask):
  rd = pltpu.make_async_copy(
      out_ref.at[:, pl.ds(start, width)], regbuf.at[:, pl.ds(0, width)], sem)
  rd.start()
  rd.wait()
  regbuf[:, pl.ds(0, width)] = jnp.where(
      mask, rolled, regbuf[:, pl.ds(0, width)])
  wr = pltpu.make_async_copy(
      regbuf.at[:, pl.ds(0, width)], out_ref.at[:, pl.ds(start, width)], sem)
  wr.start()
  wr.wait()


def _tc_body(ptr_ref, zT_ref, bank_in_ref, outT_ref, znbuf, regbuf, w_sem):
  del bank_in_ref
  zt = zT_ref[...]
  norm = jnp.sqrt(jnp.sum(zt * zt, axis=0, keepdims=True))
  znbuf[:, pl.ds(0, BATCH)] = zt / jnp.maximum(norm, 1e-12)

  s = jnp.remainder(ptr_ref[0], SIZE)
  lane = lax.broadcasted_iota(jnp.int32, (DIM, REG), 1)
  no_wrap = s + BATCH <= SIZE

  @pl.when(no_wrap)
  def _():
    a = jnp.minimum((s // 128) * 128, ANCHOR_CAP)
    a = pl.multiple_of(a, 128)
    r = s - a
    rolled = pltpu.roll(znbuf[...], r, axis=1)
    mask = jnp.logical_and(lane >= r, lane < r + BATCH)
    _merge_region(outT_ref, regbuf, w_sem, a, REG, rolled, mask)

  @pl.when(jnp.logical_not(no_wrap))
  def _():
    r_t = s - ANCHOR_CAP
    rolled_t = pltpu.roll(znbuf[...], jnp.remainder(r_t, REG), axis=1)
    mask_t = lane >= r_t
    _merge_region(outT_ref, regbuf, w_sem, ANCHOR_CAP, REG, rolled_t, mask_t)
    b1 = s + BATCH - SIZE
    rolled_h = pltpu.roll(znbuf[:, pl.ds(0, BATCH)], b1, axis=1)
    mask_h = lane[:, :BATCH] < b1
    _merge_region(outT_ref, regbuf, w_sem, 0, BATCH, rolled_h, mask_h)

def _edge_body(ptr_ref, zT_ref, in_ref, out_ref):
  s = jnp.remainder(ptr_ref[0], SIZE)
  se = s - TILE_END
  zt = zT_ref[...]
  norm = jnp.sqrt(jnp.sum(zt * zt, axis=0, keepdims=True))
  zn = zt / jnp.maximum(norm, 1e-12)
  rolled = pltpu.roll(zn, jnp.remainder(se, BATCH), axis=1)[:, :128]
  lane = lax.broadcasted_iota(jnp.int32, (DIM, 128), 1)
  mask = jnp.logical_and(lane >= se, lane < se + BATCH)
  out_ref[...] = jnp.where(mask, rolled, in_ref[...])


def kernel(z, bank, ptr):
  zT = z.T
  bankT = bank.T
  bank_copy = pl.pallas_call(
      _copy_body,
      grid=(COPY_BLOCKS,),
      in_specs=[pl.BlockSpec((DIM, COPY_W), lambda i: (0, i))],
      out_specs=pl.BlockSpec((DIM, COPY_W), lambda i: (0, i)),
      out_shape=jax.ShapeDtypeStruct((DIM, SIZE), jnp.float32),
      name="bank_copy",
  )(bankT)
  outT = pl.pallas_call(
      _tc_body,
      in_specs=[
          pl.BlockSpec(memory_space=pltpu.SMEM),
          pl.BlockSpec(memory_space=pltpu.VMEM),
          pl.BlockSpec(memory_space=pl.ANY),
      ],
      out_specs=pl.BlockSpec(memory_space=pl.ANY),
      out_shape=jax.ShapeDtypeStruct((DIM, SIZE), jnp.float32),
      input_output_aliases={2: 0},
      scratch_shapes=[
          pltpu.VMEM((DIM, REG), jnp.float32),
          pltpu.VMEM((DIM, REG), jnp.float32),
          pltpu.SemaphoreType.DMA,
      ],
      name="bank_window_write",
  )(ptr, zT, bank_copy)
  outT = pl.pallas_call(
      _edge_body,
      grid=(1,),
      in_specs=[
          pl.BlockSpec(memory_space=pltpu.SMEM),
          pl.BlockSpec((DIM, BATCH), lambda i: (0, 0)),
          pl.BlockSpec((DIM, 128), lambda i: (0, TILE_END // 128)),
      ],
      out_specs=pl.BlockSpec((DIM, 128), lambda i: (0, TILE_END // 128)),
      out_shape=jax.ShapeDtypeStruct((DIM, SIZE), jnp.float32),
      input_output_aliases={2: 0},
      name="bank_edge_fix",
  )(ptr, zT, outT)
  new_bank = outT.T
  p = ptr[0]
  new_ptr = (p + BATCH) % SIZE
  wrapped = jnp.logical_or(new_ptr < p, p + BATCH >= SIZE)
  return new_bank, jnp.array([new_ptr], dtype=jnp.int32), jnp.reshape(wrapped, (1,))
